# baseline (speedup 1.0000x reference)
import functools

import jax
import jax.numpy as jnp
from jax import lax
from jax.experimental import pallas as pl
from jax.experimental.pallas import tpu as pltpu
from jax.experimental.pallas import tpu_sc as plsc

K = 20
EPS = 1e-5


def _gelu(x):
    return 0.5 * x * (1.0 + lax.erf(x * 0.7071067811865476))


def _ln(x, g, b):
    m = jnp.mean(x, -1, keepdims=True)
    v = jnp.mean((x - m) ** 2, -1, keepdims=True)
    return g * (x - m) / jnp.sqrt(v + EPS) + b


def _inorm(x, g, b):
    m = jnp.mean(x, 0, keepdims=True)
    v = jnp.mean((x - m) ** 2, 0, keepdims=True)
    return g * (x - m) / jnp.sqrt(v + EPS) + b


def _dot(a, b, **kw):
    return jnp.dot(a, b, preferred_element_type=jnp.float32, **kw)


def _dot_t(a, b):
    return lax.dot_general(a, b, (((1,), (1,)), ((), ())),
                           preferred_element_type=jnp.float32)


def _topk_idx(rows, allp, base):
    rt, n = rows.shape[0], allp.shape[0]
    dt = _dot_t(rows, allp)
    rx2 = jnp.sum(rows * rows, -1, keepdims=True)
    ax2 = jnp.sum(allp * allp, -1)[None, :]
    d2 = jnp.maximum((rx2 + ax2) - 2.0 * dt, 0.0)

    iota = lax.broadcasted_iota(jnp.int32, (rt, n), 1)
    big_i = jnp.int32(n)
    inf = jnp.float32(3.0e38)
    cols = []
    for _ in range(K):
        m = jnp.min(d2, axis=-1, keepdims=True)
        cand = jnp.where(d2 == m, iota, big_i)
        j = jnp.min(cand, axis=-1, keepdims=True)
        cols.append(j)
        d2 = jnp.where(iota == j, inf, d2)
    return jnp.concatenate(cols, axis=1) + base


def _knn_body(n_total, rows_ref, all_ref, idx_ref):
    b = pl.program_id(0)
    idx_ref[0] = _topk_idx(rows_ref[0], all_ref[0], b * n_total)


def _knn(pts, rt=256):
    B, N, Dp = pts.shape
    return pl.pallas_call(
        functools.partial(_knn_body, N),
        grid=(B, N // rt),
        in_specs=[
            pl.BlockSpec((1, rt, Dp), lambda b, t: (b, t, 0)),
            pl.BlockSpec((1, N, Dp), lambda b, t: (b, 0, 0)),
        ],
        out_specs=pl.BlockSpec((1, rt, K), lambda b, t: (b, t, 0)),
        out_shape=jax.ShapeDtypeStruct((B, N, K), jnp.int32),
    )(pts, pts)


def _sc_gather(table, idx2d):
    nrows, D = idx2d.shape[0], table.shape[1]
    NW = 32
    CHUNK = 4
    rows_per_w = nrows // NW
    groups = rows_per_w // CHUNK
    E = nrows * 128
    mesh = plsc.VectorSubcoreMesh(core_axis_name="c", subcore_axis_name="s")

    @functools.partial(
        pl.kernel, mesh=mesh,
        out_type=jax.ShapeDtypeStruct((E, D), jnp.float32),
        scratch_types=[
            pltpu.VMEM((rows_per_w, 128), jnp.int32),
            pltpu.VMEM((CHUNK * 128, D), jnp.float32),
            pltpu.SemaphoreType.DMA,
        ],
    )
    def gath(table_hbm, idx_hbm, out_hbm, idx_v, rows_v, sem):
        wid = lax.axis_index("s") * 2 + lax.axis_index("c")
        rbase = pl.multiple_of(wid * rows_per_w, 8)
        pltpu.sync_copy(idx_hbm.at[pl.ds(rbase, rows_per_w)], idx_v)
        ebase = pl.multiple_of(wid * (rows_per_w * 128), CHUNK * 128)
        for g in range(groups):
            handles = []
            for c in range(CHUNK):
                handles.append(pltpu.async_copy(
                    table_hbm.at[idx_v.at[g * CHUNK + c]],
                    rows_v.at[pl.ds(c * 128, 128)], sem))
            for h in handles:
                h.wait()
            pltpu.sync_copy(
                rows_v, out_hbm.at[pl.ds(ebase + g * (CHUNK * 128), CHUNK * 128)])

    return gath(table, idx2d)


def _edge2_body(g_ref, x_ref, w_ref, go_ref, bo_ref, out_ref):
    X = x_ref[:, 0:64]
    w = w_ref[...]
    acc = None
    for k in range(K):
        s = jnp.concatenate([g_ref[:, k, 0:64] - X, X], axis=-1)
        e = _dot(s, w)
        acc = e if acc is None else jnp.maximum(acc, e)
    out_ref[...] = _gelu(_ln(acc, go_ref[...], bo_ref[...]))


def _edge2(G, X, w2, go, bo, rg=512):
    BN = X.shape[0]
    return pl.pallas_call(
        _edge2_body,
        grid=(BN // rg,),
        in_specs=[
            pl.BlockSpec((rg, K, 128), lambda i: (i, 0, 0)),
            pl.BlockSpec((rg, 128), lambda i: (i, 0)),
            pl.BlockSpec((128, 64), lambda i: (0, 0)),
            pl.BlockSpec((64,), lambda i: (0,)),
            pl.BlockSpec((64,), lambda i: (0,)),
        ],
        out_specs=pl.BlockSpec((rg, 64), lambda i: (i, 0)),
        out_shape=jax.ShapeDtypeStruct((BN, 64), jnp.float32),
    )(G, X, w2, go, bo)


def _wlin_body(x1_ref, x2_ref, x3_ref, wl_ref, out_ref):
    z = (_dot(x1_ref[:, 0:64], wl_ref[0:64])
         + _dot(x2_ref[:, 0:64], wl_ref[64:128])
         + _dot(x3_ref[...], wl_ref[128:192]))
    out_ref[0] = jnp.max(z, axis=0, keepdims=True)


def _wlin(x1p, x2p, x3, wl, B, N):
    return pl.pallas_call(
        _wlin_body,
        grid=(B,),
        in_specs=[
            pl.BlockSpec((N, 128), lambda b: (b, 0)),
            pl.BlockSpec((N, 128), lambda b: (b, 0)),
            pl.BlockSpec((N, 64), lambda b: (b, 0)),
            pl.BlockSpec((192, 1024), lambda b: (0, 0)),
        ],
        out_specs=pl.BlockSpec((1, 1, 1024), lambda b: (b, 0, 0)),
        out_shape=jax.ShapeDtypeStruct((B, 1, 1024), jnp.float32),
    )(x1p, x2p, x3, wl)


def _head_body(e_ref, x1_ref, x2_ref, x3_ref, ge_ref, be_ref,
               wm1_ref, gm1_ref, bm1_ref, wm2_ref, gm2_ref, bm2_ref,
               wm3_ref, gm3_ref, bm3_ref, wh_ref, bh_ref, out_ref):
    e = _gelu(_ln(e_ref[0], ge_ref[...], be_ref[...]))
    te = _dot(e, wm1_ref[0:1024])
    h = (_dot(x1_ref[:, 0:64], wm1_ref[1024:1088])
         + _dot(x2_ref[:, 0:64], wm1_ref[1088:1152])
         + _dot(x3_ref[...], wm1_ref[1152:1216])
         + te)
    h = _gelu(_inorm(h, gm1_ref[...], bm1_ref[...]))
    h = _gelu(_inorm(_dot(h, wm2_ref[...]), gm2_ref[...], bm2_ref[...]))
    h = _gelu(_inorm(_dot(h, wm3_ref[...]), gm3_ref[...], bm3_ref[...]))
    out_ref[0] = _dot(h, wh_ref[...]) + bh_ref[...]


def _head(e_raw, x1p, x2p, x3, gE, bE, Wm1, gm1, bm1, Wm2, gm2, bm2,
          Wm3, gm3, bm3, Wh, bh, B, N):
    def full_spec(a):
        nd = a.ndim
        return pl.BlockSpec(a.shape, lambda b, _nd=nd: (0,) * _nd)
    return pl.pallas_call(
        _head_body,
        grid=(B,),
        in_specs=[
            pl.BlockSpec((1, 1, 1024), lambda b: (b, 0, 0)),
            pl.BlockSpec((N, 128), lambda b: (b, 0)),
            pl.BlockSpec((N, 128), lambda b: (b, 0)),
            pl.BlockSpec((N, 64), lambda b: (b, 0)),
            full_spec(gE), full_spec(bE), full_spec(Wm1), full_spec(gm1),
            full_spec(bm1), full_spec(Wm2), full_spec(gm2), full_spec(bm2),
            full_spec(Wm3), full_spec(gm3), full_spec(bm3),
            full_spec(Wh), full_spec(bh),
        ],
        out_specs=pl.BlockSpec((1, N, 50), lambda b: (b, 0, 0)),
        out_shape=jax.ShapeDtypeStruct((B, N, 50), jnp.float32),
    )(e_raw, x1p, x2p, x3, gE, bE, Wm1, gm1, bm1, Wm2, gm2, bm2,
      Wm3, gm3, bm3, Wh, bh)


def kernel(x, xyz, W0a, g0a, b0a, W0b, g0o, b0o, W1a, g1a, b1a, W1b, g1o, b1o,
           W2, g2o, b2o, Wlin, gE, bE, Wm1, gm1, bm1, Wm2, gm2, bm2,
           Wm3, gm3, bm3, Wh, bh):
    B, N, _ = x.shape
    BN = B * N

    xp = jnp.pad(x, ((0, 0), (0, 0), (0, 125))).reshape(BN, 128)
    xyzp = jnp.pad(xyz, ((0, 0), (0, 0), (0, 125)))

    def edge_mlp(Gp, xs, df, Wa, ga, ba, Wb, go, bo):
        nb = Gp.reshape(B, N, K, 128)[..., :df]
        xc = jnp.broadcast_to(xs[:, :, None, :], nb.shape)
        gf = jnp.concatenate([nb - xc, xc], -1)
        h = jax.nn.gelu(_ln(gf @ Wa, ga, ba), approximate=False) @ Wb
        h = jnp.max(h, axis=2)
        return jax.nn.gelu(_ln(h, go, bo), approximate=False)

    idx0 = _knn(xyzp)
    G0 = _sc_gather(xp, idx0.reshape(-1, 128))
    x1 = edge_mlp(G0, x, 3, W0a, g0a, b0a, W0b, g0o, b0o)
    x1p = jnp.pad(x1, ((0, 0), (0, 0), (0, 64))).reshape(BN, 128)

    idx1 = _knn(x1p.reshape(B, N, 128))
    G1 = _sc_gather(x1p, idx1.reshape(-1, 128))
    x2 = edge_mlp(G1, x1, 64, W1a, g1a, b1a, W1b, g1o, b1o)
    x2p = jnp.pad(x2, ((0, 0), (0, 0), (0, 64))).reshape(BN, 128)

    idx2 = _knn(x2p.reshape(B, N, 128))
    G2 = _sc_gather(x2p, idx2.reshape(-1, 128))
    x3 = _edge2(G2.reshape(BN, K, 128), x2p, W2, g2o, b2o)

    e_raw = _wlin(x1p, x2p, x3, Wlin, B, N)
    out = _head(e_raw, x1p, x2p, x3, gE, bE,
                Wm1, gm1.reshape(-1), bm1.reshape(-1),
                Wm2, gm2.reshape(-1), bm2.reshape(-1),
                Wm3, gm3.reshape(-1), bm3.reshape(-1), Wh, bh, B, N)
    return out


# speedup vs baseline: 1.0629x; 1.0629x over previous
import functools

import jax
import jax.numpy as jnp
from jax import lax
from jax.experimental import pallas as pl
from jax.experimental.pallas import tpu as pltpu
from jax.experimental.pallas import tpu_sc as plsc

K = 20
EPS = 1e-5


def _gelu(x):
    return 0.5 * x * (1.0 + lax.erf(x * 0.7071067811865476))


def _ln(x, g, b):
    m = jnp.mean(x, -1, keepdims=True)
    v = jnp.mean((x - m) ** 2, -1, keepdims=True)
    return g * (x - m) / jnp.sqrt(v + EPS) + b


def _inorm(x, g, b):
    m = jnp.mean(x, 0, keepdims=True)
    v = jnp.mean((x - m) ** 2, 0, keepdims=True)
    return g * (x - m) / jnp.sqrt(v + EPS) + b


def _dot(a, b, **kw):
    return jnp.dot(a, b, preferred_element_type=jnp.float32, **kw)


def _dot_t(a, b):
    return lax.dot_general(a, b, (((1,), (1,)), ((), ())),
                           preferred_element_type=jnp.float32)


def _topk_idx(rows, allp, base):
    rt, n = rows.shape[0], allp.shape[0]
    dt = _dot_t(rows, allp)
    rx2 = jnp.sum(rows * rows, -1, keepdims=True)
    ax2 = jnp.sum(allp * allp, -1)[None, :]
    d2 = jnp.maximum((rx2 + ax2) - 2.0 * dt, 0.0)

    iota = lax.broadcasted_iota(jnp.int32, (rt, n), 1)
    big_i = jnp.int32(n)
    inf = jnp.float32(3.0e38)
    cols = []
    for _ in range(K):
        m = jnp.min(d2, axis=-1, keepdims=True)
        cand = jnp.where(d2 == m, iota, big_i)
        j = jnp.min(cand, axis=-1, keepdims=True)
        cols.append(j)
        d2 = jnp.where(iota == j, inf, d2)
    return jnp.concatenate(cols, axis=1) + base


def _knn_body(n_total, rows_ref, all_ref, idx_ref):
    b = pl.program_id(0)
    idx_ref[0] = _topk_idx(rows_ref[0], all_ref[0], b * n_total)


def _knn(pts, rt=512):
    B, N, Dp = pts.shape
    return pl.pallas_call(
        functools.partial(_knn_body, N),
        grid=(B, N // rt),
        in_specs=[
            pl.BlockSpec((1, rt, Dp), lambda b, t: (b, t, 0)),
            pl.BlockSpec((1, N, Dp), lambda b, t: (b, 0, 0)),
        ],
        out_specs=pl.BlockSpec((1, rt, K), lambda b, t: (b, t, 0)),
        out_shape=jax.ShapeDtypeStruct((B, N, K), jnp.int32),
    )(pts, pts)


def _sc_gather(table, idx2d):
    nrows, D = idx2d.shape[0], table.shape[1]
    NW = 32
    CHUNK = 4
    rows_per_w = nrows // NW
    groups = rows_per_w // CHUNK
    E = nrows * 128
    mesh = plsc.VectorSubcoreMesh(core_axis_name="c", subcore_axis_name="s")

    @functools.partial(
        pl.kernel, mesh=mesh,
        out_type=jax.ShapeDtypeStruct((E, D), jnp.float32),
        scratch_types=[
            pltpu.VMEM((rows_per_w, 128), jnp.int32),
            pltpu.VMEM((CHUNK * 128, D), jnp.float32),
            pltpu.SemaphoreType.DMA,
        ],
    )
    def gath(table_hbm, idx_hbm, out_hbm, idx_v, rows_v, sem):
        wid = lax.axis_index("s") * 2 + lax.axis_index("c")
        rbase = pl.multiple_of(wid * rows_per_w, 8)
        pltpu.sync_copy(idx_hbm.at[pl.ds(rbase, rows_per_w)], idx_v)
        ebase = pl.multiple_of(wid * (rows_per_w * 128), CHUNK * 128)
        for g in range(groups):
            handles = []
            for c in range(CHUNK):
                handles.append(pltpu.async_copy(
                    table_hbm.at[idx_v.at[g * CHUNK + c]],
                    rows_v.at[pl.ds(c * 128, 128)], sem))
            for h in handles:
                h.wait()
            pltpu.sync_copy(
                rows_v, out_hbm.at[pl.ds(ebase + g * (CHUNK * 128), CHUNK * 128)])

    return gath(table, idx2d)


def _edge2_body(g_ref, x_ref, w_ref, go_ref, bo_ref, out_ref):
    X = x_ref[:, 0:64]
    w = w_ref[...]
    acc = None
    for k in range(K):
        s = jnp.concatenate([g_ref[:, k, 0:64] - X, X], axis=-1)
        e = _dot(s, w)
        acc = e if acc is None else jnp.maximum(acc, e)
    out_ref[...] = _gelu(_ln(acc, go_ref[...], bo_ref[...]))


def _edge2(G, X, w2, go, bo, rg=512):
    BN = X.shape[0]
    return pl.pallas_call(
        _edge2_body,
        grid=(BN // rg,),
        in_specs=[
            pl.BlockSpec((rg, K, 128), lambda i: (i, 0, 0)),
            pl.BlockSpec((rg, 128), lambda i: (i, 0)),
            pl.BlockSpec((128, 64), lambda i: (0, 0)),
            pl.BlockSpec((64,), lambda i: (0,)),
            pl.BlockSpec((64,), lambda i: (0,)),
        ],
        out_specs=pl.BlockSpec((rg, 64), lambda i: (i, 0)),
        out_shape=jax.ShapeDtypeStruct((BN, 64), jnp.float32),
    )(G, X, w2, go, bo)


def _wlin_body(x1_ref, x2_ref, x3_ref, wl_ref, out_ref):
    z = (_dot(x1_ref[:, 0:64], wl_ref[0:64])
         + _dot(x2_ref[:, 0:64], wl_ref[64:128])
         + _dot(x3_ref[...], wl_ref[128:192]))
    out_ref[0] = jnp.max(z, axis=0, keepdims=True)


def _wlin(x1p, x2p, x3, wl, B, N):
    return pl.pallas_call(
        _wlin_body,
        grid=(B,),
        in_specs=[
            pl.BlockSpec((N, 128), lambda b: (b, 0)),
            pl.BlockSpec((N, 128), lambda b: (b, 0)),
            pl.BlockSpec((N, 64), lambda b: (b, 0)),
            pl.BlockSpec((192, 1024), lambda b: (0, 0)),
        ],
        out_specs=pl.BlockSpec((1, 1, 1024), lambda b: (b, 0, 0)),
        out_shape=jax.ShapeDtypeStruct((B, 1, 1024), jnp.float32),
    )(x1p, x2p, x3, wl)


def _head_body(e_ref, x1_ref, x2_ref, x3_ref, ge_ref, be_ref,
               wm1_ref, gm1_ref, bm1_ref, wm2_ref, gm2_ref, bm2_ref,
               wm3_ref, gm3_ref, bm3_ref, wh_ref, bh_ref, out_ref):
    e = _gelu(_ln(e_ref[0], ge_ref[...], be_ref[...]))
    te = _dot(e, wm1_ref[0:1024])
    h = (_dot(x1_ref[:, 0:64], wm1_ref[1024:1088])
         + _dot(x2_ref[:, 0:64], wm1_ref[1088:1152])
         + _dot(x3_ref[...], wm1_ref[1152:1216])
         + te)
    h = _gelu(_inorm(h, gm1_ref[...], bm1_ref[...]))
    h = _gelu(_inorm(_dot(h, wm2_ref[...]), gm2_ref[...], bm2_ref[...]))
    h = _gelu(_inorm(_dot(h, wm3_ref[...]), gm3_ref[...], bm3_ref[...]))
    out_ref[0] = _dot(h, wh_ref[...]) + bh_ref[...]


def _head(e_raw, x1p, x2p, x3, gE, bE, Wm1, gm1, bm1, Wm2, gm2, bm2,
          Wm3, gm3, bm3, Wh, bh, B, N):
    def full_spec(a):
        nd = a.ndim
        return pl.BlockSpec(a.shape, lambda b, _nd=nd: (0,) * _nd)
    return pl.pallas_call(
        _head_body,
        grid=(B,),
        in_specs=[
            pl.BlockSpec((1, 1, 1024), lambda b: (b, 0, 0)),
            pl.BlockSpec((N, 128), lambda b: (b, 0)),
            pl.BlockSpec((N, 128), lambda b: (b, 0)),
            pl.BlockSpec((N, 64), lambda b: (b, 0)),
            full_spec(gE), full_spec(bE), full_spec(Wm1), full_spec(gm1),
            full_spec(bm1), full_spec(Wm2), full_spec(gm2), full_spec(bm2),
            full_spec(Wm3), full_spec(gm3), full_spec(bm3),
            full_spec(Wh), full_spec(bh),
        ],
        out_specs=pl.BlockSpec((1, N, 50), lambda b: (b, 0, 0)),
        out_shape=jax.ShapeDtypeStruct((B, N, 50), jnp.float32),
    )(e_raw, x1p, x2p, x3, gE, bE, Wm1, gm1, bm1, Wm2, gm2, bm2,
      Wm3, gm3, bm3, Wh, bh)


def kernel(x, xyz, W0a, g0a, b0a, W0b, g0o, b0o, W1a, g1a, b1a, W1b, g1o, b1o,
           W2, g2o, b2o, Wlin, gE, bE, Wm1, gm1, bm1, Wm2, gm2, bm2,
           Wm3, gm3, bm3, Wh, bh):
    B, N, _ = x.shape
    BN = B * N

    xp = jnp.pad(x, ((0, 0), (0, 0), (0, 125))).reshape(BN, 128)
    xyzp = jnp.pad(xyz, ((0, 0), (0, 0), (0, 125)))

    def edge_mlp(Gp, xs, df, Wa, ga, ba, Wb, go, bo):
        nb = Gp.reshape(B, N, K, 128)[..., :df]
        xc = jnp.broadcast_to(xs[:, :, None, :], nb.shape)
        gf = jnp.concatenate([nb - xc, xc], -1)
        h = jax.nn.gelu(_ln(gf @ Wa, ga, ba), approximate=False) @ Wb
        h = jnp.max(h, axis=2)
        return jax.nn.gelu(_ln(h, go, bo), approximate=False)

    idx0 = _knn(xyzp)
    G0 = _sc_gather(xp, idx0.reshape(-1, 128))
    x1 = edge_mlp(G0, x, 3, W0a, g0a, b0a, W0b, g0o, b0o)
    x1p = jnp.pad(x1, ((0, 0), (0, 0), (0, 64))).reshape(BN, 128)

    idx1 = _knn(x1p.reshape(B, N, 128))
    G1 = _sc_gather(x1p, idx1.reshape(-1, 128))
    x2 = edge_mlp(G1, x1, 64, W1a, g1a, b1a, W1b, g1o, b1o)
    x2p = jnp.pad(x2, ((0, 0), (0, 0), (0, 64))).reshape(BN, 128)

    idx2 = _knn(x2p.reshape(B, N, 128))
    G2 = _sc_gather(x2p, idx2.reshape(-1, 128))
    x3 = _edge2(G2.reshape(BN, K, 128), x2p, W2, g2o, b2o)

    e_raw = _wlin(x1p, x2p, x3, Wlin, B, N)
    out = _head(e_raw, x1p, x2p, x3, gE, bE,
                Wm1, gm1.reshape(-1), bm1.reshape(-1),
                Wm2, gm2.reshape(-1), bm2.reshape(-1),
                Wm3, gm3.reshape(-1), bm3.reshape(-1), Wh, bh, B, N)
    return out
